# Initial kernel scaffold; baseline (speedup 1.0000x reference)
import jax
import jax.numpy as jnp
from jax.experimental import pallas as pl


def kernel(gene_id, disease_id, explicit_features, gene_table, disease_table, W1, b1, gamma, beta, W2, b2, W3, b3):
    raise NotImplementedError("write your pallas kernel here")


# speedup vs baseline: 1.2691x; 1.2691x over previous
import functools

import jax
import jax.numpy as jnp
from jax import lax
from jax.experimental import pallas as pl
from jax.experimental.pallas import tpu as pltpu
from jax.experimental.pallas import tpu_sc as plsc

BATCH = 16384
EMBED_DIM = 64
NUM_FEATURES = 128

NUM_CORES = 2
NUM_SUBCORES = 16
NUM_WORKERS = NUM_CORES * NUM_SUBCORES
B_PER_W = BATCH // NUM_WORKERS
IDX_CHUNK = 128
N_CHUNKS = B_PER_W // IDX_CHUNK


def _gather_body(gene_tab, dis_tab, gid_hbm, did_hbm, g_out, d_out,
                 idx_g, idx_d, rows_g, rows_d, sem):
    wid = lax.axis_index("s") * NUM_CORES + lax.axis_index("c")
    base = wid * B_PER_W
    pltpu.sync_copy(gid_hbm.at[wid], idx_g)
    pltpu.sync_copy(did_hbm.at[wid], idx_d)
    copies = []
    for j in range(N_CHUNKS):
        copies.append(pltpu.async_copy(
            gene_tab.at[idx_g.at[j]],
            rows_g.at[pl.ds(j * IDX_CHUNK, IDX_CHUNK)], sem))
        copies.append(pltpu.async_copy(
            dis_tab.at[idx_d.at[j]],
            rows_d.at[pl.ds(j * IDX_CHUNK, IDX_CHUNK)], sem))
    for c in copies:
        c.wait()
    pltpu.sync_copy(rows_g, g_out.at[pl.ds(base, B_PER_W)])
    pltpu.sync_copy(rows_d, d_out.at[pl.ds(base, B_PER_W)])


def _sc_gather(gene_table, disease_table, gid, did):
    mesh = plsc.VectorSubcoreMesh(core_axis_name="c", subcore_axis_name="s")
    out_type = (
        jax.ShapeDtypeStruct((BATCH, EMBED_DIM), jnp.float32),
        jax.ShapeDtypeStruct((BATCH, EMBED_DIM), jnp.float32),
    )
    scratch = [
        pltpu.VMEM((N_CHUNKS, IDX_CHUNK), jnp.int32),
        pltpu.VMEM((N_CHUNKS, IDX_CHUNK), jnp.int32),
        pltpu.VMEM((B_PER_W, EMBED_DIM), jnp.float32),
        pltpu.VMEM((B_PER_W, EMBED_DIM), jnp.float32),
        pltpu.SemaphoreType.DMA,
    ]
    run = pl.kernel(_gather_body, out_type=out_type, mesh=mesh,
                    scratch_types=scratch,
                    compiler_params=pltpu.CompilerParams(
                        use_tc_tiling_on_sc=False))
    return run(gene_table, disease_table, gid, did)


def _mlp_body(g_ref, d_ref, x_ref, w1g_ref, w1d_ref, w1x_ref, b1_ref,
              gamma_ref, beta_ref, w2_ref, b2_ref, w3_ref, b3_ref, out_ref):
    h = (jnp.dot(g_ref[...], w1g_ref[...], preferred_element_type=jnp.float32)
         + jnp.dot(d_ref[...], w1d_ref[...], preferred_element_type=jnp.float32)
         + jnp.dot(x_ref[...], w1x_ref[...], preferred_element_type=jnp.float32)
         + b1_ref[...])
    mean = jnp.mean(h, axis=0, keepdims=True)
    cent = h - mean
    var = jnp.mean(cent * cent, axis=0, keepdims=True)
    h = cent * lax.rsqrt(var + 1e-5) * gamma_ref[...] + beta_ref[...]
    h = jnp.maximum(h, 0.0)
    h2 = jnp.maximum(
        jnp.dot(h, w2_ref[...], preferred_element_type=jnp.float32)
        + b2_ref[...], 0.0)
    z = jnp.dot(h2, w3_ref[...], preferred_element_type=jnp.float32) + b3_ref[...]
    out_ref[...] = jax.nn.sigmoid(z)


def _tc_mlp(g_emb, d_emb, x, w1g, w1d, w1x, b1, gamma, beta, w2, b2, w3, b3):
    return pl.pallas_call(
        _mlp_body,
        out_shape=jax.ShapeDtypeStruct((BATCH, 1), jnp.float32),
    )(g_emb, d_emb, x, w1g, w1d, w1x, b1, gamma, beta, w2, b2, w3, b3)


def kernel(gene_id, disease_id, explicit_features, gene_table, disease_table,
           W1, b1, gamma, beta, W2, b2, W3, b3):
    gid = gene_id.astype(jnp.int32).reshape(NUM_WORKERS, N_CHUNKS, IDX_CHUNK)
    did = disease_id.astype(jnp.int32).reshape(NUM_WORKERS, N_CHUNKS, IDX_CHUNK)
    g_emb, d_emb = _sc_gather(gene_table, disease_table, gid, did)
    w1g = W1[:EMBED_DIM]
    w1d = W1[EMBED_DIM:2 * EMBED_DIM]
    w1x = W1[2 * EMBED_DIM:]
    return _tc_mlp(g_emb, d_emb, explicit_features,
                   w1g, w1d, w1x,
                   b1.reshape(1, -1), gamma.reshape(1, -1),
                   beta.reshape(1, -1), W2, b2.reshape(1, -1),
                   W3, b3.reshape(1, -1))


# speedup vs baseline: 1.2819x; 1.0101x over previous
import functools

import jax
import jax.numpy as jnp
from jax import lax
from jax.experimental import pallas as pl
from jax.experimental.pallas import tpu as pltpu
from jax.experimental.pallas import tpu_sc as plsc

BATCH = 16384
EMBED_DIM = 64
NUM_FEATURES = 128

NUM_CORES = 2
NUM_SUBCORES = 16
NUM_WORKERS = NUM_CORES * NUM_SUBCORES
B_PER_W = BATCH // NUM_WORKERS
IDX_CHUNK = 128
N_CHUNKS = B_PER_W // IDX_CHUNK


def _gather_body(gene_tab, dis_tab, gid_hbm, did_hbm, g_out, d_out,
                 idx_g, idx_d, rows_g, rows_d, sem):
    wid = lax.axis_index("s") * NUM_CORES + lax.axis_index("c")
    base = wid * B_PER_W
    pltpu.sync_copy(gid_hbm.at[wid], idx_g)
    pltpu.sync_copy(did_hbm.at[wid], idx_d)
    copies = []
    for j in range(N_CHUNKS):
        copies.append(pltpu.async_copy(
            gene_tab.at[idx_g.at[j]],
            rows_g.at[pl.ds(j * IDX_CHUNK, IDX_CHUNK)], sem))
        copies.append(pltpu.async_copy(
            dis_tab.at[idx_d.at[j]],
            rows_d.at[pl.ds(j * IDX_CHUNK, IDX_CHUNK)], sem))
    for c in copies:
        c.wait()
    pltpu.sync_copy(rows_g, g_out.at[pl.ds(base, B_PER_W)])
    pltpu.sync_copy(rows_d, d_out.at[pl.ds(base, B_PER_W)])


def _sc_gather(gene_table, disease_table, gid, did):
    mesh = plsc.VectorSubcoreMesh(core_axis_name="c", subcore_axis_name="s")
    out_type = (
        jax.ShapeDtypeStruct((BATCH, EMBED_DIM), jnp.float32),
        jax.ShapeDtypeStruct((BATCH, EMBED_DIM), jnp.float32),
    )
    scratch = [
        pltpu.VMEM((N_CHUNKS, IDX_CHUNK), jnp.int32),
        pltpu.VMEM((N_CHUNKS, IDX_CHUNK), jnp.int32),
        pltpu.VMEM((B_PER_W, EMBED_DIM), jnp.float32),
        pltpu.VMEM((B_PER_W, EMBED_DIM), jnp.float32),
        pltpu.SemaphoreType.DMA,
    ]
    run = pl.kernel(_gather_body, out_type=out_type, mesh=mesh,
                    scratch_types=scratch,
                    compiler_params=pltpu.CompilerParams(
                        use_tc_tiling_on_sc=False))
    return run(gene_table, disease_table, gid, did)


BLK = 2048
N_BLK = BATCH // BLK


def _pass1_body(g_ref, d_ref, x_ref, w1g_ref, w1d_ref, w1x_ref, b1_ref,
                h_ref, stats_ref):
    i = pl.program_id(0)
    h = (jnp.dot(g_ref[...], w1g_ref[...], preferred_element_type=jnp.float32)
         + jnp.dot(d_ref[...], w1d_ref[...], preferred_element_type=jnp.float32)
         + jnp.dot(x_ref[...], w1x_ref[...], preferred_element_type=jnp.float32)
         + b1_ref[...])
    h_ref[...] = h
    part = jnp.concatenate(
        [jnp.sum(h, axis=0, keepdims=True),
         jnp.sum(h * h, axis=0, keepdims=True)], axis=0)

    @pl.when(i == 0)
    def _():
        stats_ref[...] = part

    @pl.when(i != 0)
    def _():
        stats_ref[...] += part


def _pass2_body(h_ref, stats_ref, gamma_ref, beta_ref, w2_ref, b2_ref,
                w3_ref, b3_ref, out_ref):
    inv_n = 1.0 / BATCH
    mean = stats_ref[0:1, :] * inv_n
    var = stats_ref[1:2, :] * inv_n - mean * mean
    scale = lax.rsqrt(var + 1e-5) * gamma_ref[...]
    shift = beta_ref[...] - mean * scale
    h = jnp.maximum(h_ref[...] * scale + shift, 0.0)
    h2 = jnp.maximum(
        jnp.dot(h, w2_ref[...], preferred_element_type=jnp.float32)
        + b2_ref[...], 0.0)
    z = jnp.dot(h2, w3_ref[...], preferred_element_type=jnp.float32) + b3_ref[...]
    out_ref[...] = jax.nn.sigmoid(z)


def _tc_mlp(g_emb, d_emb, x, w1g, w1d, w1x, b1, gamma, beta, w2, b2, w3, b3):
    row_blk = lambda i: (i, 0)
    fixed = lambda i: (0, 0)
    h, stats = pl.pallas_call(
        _pass1_body,
        grid=(N_BLK,),
        in_specs=[
            pl.BlockSpec((BLK, EMBED_DIM), row_blk),
            pl.BlockSpec((BLK, EMBED_DIM), row_blk),
            pl.BlockSpec((BLK, NUM_FEATURES), row_blk),
            pl.BlockSpec((EMBED_DIM, 128), fixed),
            pl.BlockSpec((EMBED_DIM, 128), fixed),
            pl.BlockSpec((NUM_FEATURES, 128), fixed),
            pl.BlockSpec((1, 128), fixed),
        ],
        out_specs=[
            pl.BlockSpec((BLK, 128), row_blk),
            pl.BlockSpec((2, 128), fixed),
        ],
        out_shape=[
            jax.ShapeDtypeStruct((BATCH, 128), jnp.float32),
            jax.ShapeDtypeStruct((2, 128), jnp.float32),
        ],
    )(g_emb, d_emb, x, w1g, w1d, w1x, b1)
    return pl.pallas_call(
        _pass2_body,
        grid=(N_BLK,),
        in_specs=[
            pl.BlockSpec((BLK, 128), row_blk),
            pl.BlockSpec((2, 128), fixed),
            pl.BlockSpec((1, 128), fixed),
            pl.BlockSpec((1, 128), fixed),
            pl.BlockSpec((128, 64), fixed),
            pl.BlockSpec((1, 64), fixed),
            pl.BlockSpec((64, 1), fixed),
            pl.BlockSpec((1, 1), fixed),
        ],
        out_specs=pl.BlockSpec((BLK, 1), row_blk),
        out_shape=jax.ShapeDtypeStruct((BATCH, 1), jnp.float32),
    )(h, stats, gamma, beta, w2, b2, w3, b3)


def kernel(gene_id, disease_id, explicit_features, gene_table, disease_table,
           W1, b1, gamma, beta, W2, b2, W3, b3):
    gid = gene_id.astype(jnp.int32).reshape(NUM_WORKERS, N_CHUNKS, IDX_CHUNK)
    did = disease_id.astype(jnp.int32).reshape(NUM_WORKERS, N_CHUNKS, IDX_CHUNK)
    g_emb, d_emb = _sc_gather(gene_table, disease_table, gid, did)
    w1g = W1[:EMBED_DIM]
    w1d = W1[EMBED_DIM:2 * EMBED_DIM]
    w1x = W1[2 * EMBED_DIM:]
    return _tc_mlp(g_emb, d_emb, explicit_features,
                   w1g, w1d, w1x,
                   b1.reshape(1, -1), gamma.reshape(1, -1),
                   beta.reshape(1, -1), W2, b2.reshape(1, -1),
                   W3, b3.reshape(1, -1))


# speedup vs baseline: 1.4105x; 1.1003x over previous
import functools

import jax
import jax.numpy as jnp
from jax import lax
from jax.experimental import pallas as pl
from jax.experimental.pallas import tpu as pltpu
from jax.experimental.pallas import tpu_sc as plsc

BATCH = 16384
EMBED_DIM = 64
EMBED_PAD = 128
NUM_FEATURES = 128

NUM_CORES = 2
NUM_SUBCORES = 16
NUM_WORKERS = NUM_CORES * NUM_SUBCORES
B_PER_W = BATCH // NUM_WORKERS
IDX_CHUNK = 128
N_CHUNKS = B_PER_W // IDX_CHUNK


def _gather_body(gene_tab, dis_tab, gid_hbm, did_hbm, g_out, d_out,
                 idx_g, idx_d, rows, sem):
    wid = lax.axis_index("s") * NUM_CORES + lax.axis_index("c")
    base = wid * B_PER_W
    pltpu.sync_copy(gid_hbm.at[wid], idx_g)
    pltpu.sync_copy(did_hbm.at[wid], idx_d)
    copies = []
    for j in range(N_CHUNKS):
        copies.append(pltpu.async_copy(
            gene_tab.at[idx_g.at[j]],
            rows.at[pl.ds(j * IDX_CHUNK, IDX_CHUNK)], sem))
    for c in copies:
        c.wait()
    pltpu.sync_copy(rows, g_out.at[pl.ds(base, B_PER_W)])
    copies = []
    for j in range(N_CHUNKS):
        copies.append(pltpu.async_copy(
            dis_tab.at[idx_d.at[j]],
            rows.at[pl.ds(j * IDX_CHUNK, IDX_CHUNK)], sem))
    for c in copies:
        c.wait()
    pltpu.sync_copy(rows, d_out.at[pl.ds(base, B_PER_W)])


def _sc_gather(gene_table, disease_table, gid, did):
    mesh = plsc.VectorSubcoreMesh(core_axis_name="c", subcore_axis_name="s")
    out_type = (
        jax.ShapeDtypeStruct((BATCH, EMBED_PAD), jnp.float32),
        jax.ShapeDtypeStruct((BATCH, EMBED_PAD), jnp.float32),
    )
    scratch = [
        pltpu.VMEM((N_CHUNKS, IDX_CHUNK), jnp.int32),
        pltpu.VMEM((N_CHUNKS, IDX_CHUNK), jnp.int32),
        pltpu.VMEM((B_PER_W, EMBED_PAD), jnp.float32),
        pltpu.SemaphoreType.DMA,
    ]
    run = pl.kernel(_gather_body, out_type=out_type, mesh=mesh,
                    scratch_types=scratch)
    return run(gene_table, disease_table, gid, did)


BLK = 2048
N_BLK = BATCH // BLK


def _pass1_body(g_ref, d_ref, x_ref, w1g_ref, w1d_ref, w1x_ref, b1_ref,
                h_ref, stats_ref):
    i = pl.program_id(0)
    h = (jnp.dot(g_ref[...], w1g_ref[...], preferred_element_type=jnp.float32)
         + jnp.dot(d_ref[...], w1d_ref[...], preferred_element_type=jnp.float32)
         + jnp.dot(x_ref[...], w1x_ref[...], preferred_element_type=jnp.float32)
         + b1_ref[...])
    h_ref[...] = h
    part = jnp.concatenate(
        [jnp.sum(h, axis=0, keepdims=True),
         jnp.sum(h * h, axis=0, keepdims=True)], axis=0)

    @pl.when(i == 0)
    def _():
        stats_ref[...] = part

    @pl.when(i != 0)
    def _():
        stats_ref[...] += part


def _pass2_body(h_ref, stats_ref, gamma_ref, beta_ref, w2_ref, b2_ref,
                w3_ref, b3_ref, out_ref):
    inv_n = 1.0 / BATCH
    mean = stats_ref[0:1, :] * inv_n
    var = stats_ref[1:2, :] * inv_n - mean * mean
    scale = lax.rsqrt(var + 1e-5) * gamma_ref[...]
    shift = beta_ref[...] - mean * scale
    h = jnp.maximum(h_ref[...] * scale + shift, 0.0)
    h2 = jnp.maximum(
        jnp.dot(h, w2_ref[...], preferred_element_type=jnp.float32)
        + b2_ref[...], 0.0)
    z = jnp.dot(h2, w3_ref[...], preferred_element_type=jnp.float32) + b3_ref[...]
    out_ref[...] = jax.nn.sigmoid(z)


def _tc_mlp(g_emb, d_emb, x, w1g, w1d, w1x, b1, gamma, beta, w2, b2, w3, b3):
    row_blk = lambda i: (i, 0)
    fixed = lambda i: (0, 0)
    h, stats = pl.pallas_call(
        _pass1_body,
        grid=(N_BLK,),
        in_specs=[
            pl.BlockSpec((BLK, EMBED_PAD), row_blk),
            pl.BlockSpec((BLK, EMBED_PAD), row_blk),
            pl.BlockSpec((BLK, NUM_FEATURES), row_blk),
            pl.BlockSpec((EMBED_PAD, 128), fixed),
            pl.BlockSpec((EMBED_PAD, 128), fixed),
            pl.BlockSpec((NUM_FEATURES, 128), fixed),
            pl.BlockSpec((1, 128), fixed),
        ],
        out_specs=[
            pl.BlockSpec((BLK, 128), row_blk),
            pl.BlockSpec((2, 128), fixed),
        ],
        out_shape=[
            jax.ShapeDtypeStruct((BATCH, 128), jnp.float32),
            jax.ShapeDtypeStruct((2, 128), jnp.float32),
        ],
    )(g_emb, d_emb, x, w1g, w1d, w1x, b1)
    return pl.pallas_call(
        _pass2_body,
        grid=(N_BLK,),
        in_specs=[
            pl.BlockSpec((BLK, 128), row_blk),
            pl.BlockSpec((2, 128), fixed),
            pl.BlockSpec((1, 128), fixed),
            pl.BlockSpec((1, 128), fixed),
            pl.BlockSpec((128, 64), fixed),
            pl.BlockSpec((1, 64), fixed),
            pl.BlockSpec((64, 1), fixed),
            pl.BlockSpec((1, 1), fixed),
        ],
        out_specs=pl.BlockSpec((BLK, 1), row_blk),
        out_shape=jax.ShapeDtypeStruct((BATCH, 1), jnp.float32),
    )(h, stats, gamma, beta, w2, b2, w3, b3)


def kernel(gene_id, disease_id, explicit_features, gene_table, disease_table,
           W1, b1, gamma, beta, W2, b2, W3, b3):
    gid = gene_id.astype(jnp.int32).reshape(NUM_WORKERS, N_CHUNKS, IDX_CHUNK)
    did = disease_id.astype(jnp.int32).reshape(NUM_WORKERS, N_CHUNKS, IDX_CHUNK)
    pad = EMBED_PAD - EMBED_DIM
    gene_pad = jnp.pad(gene_table, ((0, 0), (0, pad)))
    dis_pad = jnp.pad(disease_table, ((0, 0), (0, pad)))
    g_emb, d_emb = _sc_gather(gene_pad, dis_pad, gid, did)
    w1g = jnp.pad(W1[:EMBED_DIM], ((0, pad), (0, 0)))
    w1d = jnp.pad(W1[EMBED_DIM:2 * EMBED_DIM], ((0, pad), (0, 0)))
    w1x = W1[2 * EMBED_DIM:]
    return _tc_mlp(g_emb, d_emb, explicit_features,
                   w1g, w1d, w1x,
                   b1.reshape(1, -1), gamma.reshape(1, -1),
                   beta.reshape(1, -1), W2, b2.reshape(1, -1),
                   W3, b3.reshape(1, -1))


# speedup vs baseline: 1.5102x; 1.0707x over previous
import functools

import jax
import jax.numpy as jnp
from jax import lax
from jax.experimental import pallas as pl
from jax.experimental.pallas import tpu as pltpu
from jax.experimental.pallas import tpu_sc as plsc

BATCH = 16384
EMBED_DIM = 64
NUM_FEATURES = 128
NUM_GENES = 100000
NUM_DISEASES = 1000
HID = 128

NUM_CORES = 2
NUM_SUBCORES = 16
NUM_WORKERS = NUM_CORES * NUM_SUBCORES
B_PER_W = BATCH // NUM_WORKERS
IDX_CHUNK = 128
N_CHUNKS = B_PER_W // IDX_CHUNK

GBLK = 2048
N_GBLK = -(-NUM_GENES // GBLK)


def _pg_body(tT_ref, w_ref, out_ref):
    out_ref[...] = lax.dot_general(
        tT_ref[...], w_ref[...],
        dimension_numbers=(((0,), (0,)), ((), ())),
        preferred_element_type=jnp.float32)


def _pd_body(tT_ref, w_ref, b_ref, out_ref):
    out_ref[...] = lax.dot_general(
        tT_ref[...], w_ref[...],
        dimension_numbers=(((0,), (0,)), ((), ())),
        preferred_element_type=jnp.float32) + b_ref[...]


def _build_products(gene_table, disease_table, W1, b1):
    gT = jnp.transpose(gene_table)
    dT = jnp.transpose(disease_table)
    w1g = W1[:EMBED_DIM]
    w1d = W1[EMBED_DIM:2 * EMBED_DIM]
    pg = pl.pallas_call(
        _pg_body,
        grid=(N_GBLK,),
        in_specs=[
            pl.BlockSpec((EMBED_DIM, GBLK), lambda i: (0, i)),
            pl.BlockSpec((EMBED_DIM, HID), lambda i: (0, 0)),
        ],
        out_specs=pl.BlockSpec((GBLK, HID), lambda i: (i, 0)),
        out_shape=jax.ShapeDtypeStruct((NUM_GENES, HID), jnp.float32),
    )(gT, w1g)
    pd = pl.pallas_call(
        _pd_body,
        out_shape=jax.ShapeDtypeStruct((NUM_DISEASES, HID), jnp.float32),
    )(dT, w1d, b1)
    return pg, pd


def _gather_body(pg_tab, pd_tab, gid_hbm, did_hbm, pre_out, idx_g, idx_d,
                 rows, sem):
    wid = lax.axis_index("s") * NUM_CORES + lax.axis_index("c")
    base = wid * B_PER_W
    pltpu.sync_copy(gid_hbm.at[wid], idx_g)
    pltpu.sync_copy(did_hbm.at[wid], idx_d)
    copies = []
    for j in range(N_CHUNKS):
        copies.append(pltpu.async_copy(
            pg_tab.at[idx_g.at[j]],
            rows.at[pl.ds(j * IDX_CHUNK, IDX_CHUNK)], sem))
    for c in copies:
        c.wait()
    copies = []
    for j in range(N_CHUNKS):
        copies.append(pltpu.async_copy(
            pd_tab.at[idx_d.at[j]],
            rows.at[pl.ds(j * IDX_CHUNK, IDX_CHUNK)], sem, add=True))
    for c in copies:
        c.wait()
    pltpu.sync_copy(rows, pre_out.at[pl.ds(base, B_PER_W)])


def _sc_gather(pg, pd, gid, did):
    mesh = plsc.VectorSubcoreMesh(core_axis_name="c", subcore_axis_name="s")
    out_type = jax.ShapeDtypeStruct((BATCH, HID), jnp.float32)
    scratch = [
        pltpu.VMEM((N_CHUNKS, IDX_CHUNK), jnp.int32),
        pltpu.VMEM((N_CHUNKS, IDX_CHUNK), jnp.int32),
        pltpu.VMEM((B_PER_W, HID), jnp.float32),
        pltpu.SemaphoreType.DMA,
    ]
    run = pl.kernel(_gather_body, out_type=out_type, mesh=mesh,
                    scratch_types=scratch)
    return run(pg, pd, gid, did)


BLK = 2048
N_BLK = BATCH // BLK


def _pass1_body(pre_ref, x_ref, w1x_ref, h_ref, stats_ref):
    i = pl.program_id(0)
    h = pre_ref[...] + jnp.dot(x_ref[...], w1x_ref[...],
                               preferred_element_type=jnp.float32)
    h_ref[...] = h
    part = jnp.concatenate(
        [jnp.sum(h, axis=0, keepdims=True),
         jnp.sum(h * h, axis=0, keepdims=True)], axis=0)

    @pl.when(i == 0)
    def _():
        stats_ref[...] = part

    @pl.when(i != 0)
    def _():
        stats_ref[...] += part


def _pass2_body(h_ref, stats_ref, gamma_ref, beta_ref, w2_ref, b2_ref,
                w3_ref, b3_ref, out_ref):
    inv_n = 1.0 / BATCH
    mean = stats_ref[0:1, :] * inv_n
    var = stats_ref[1:2, :] * inv_n - mean * mean
    scale = lax.rsqrt(var + 1e-5) * gamma_ref[...]
    shift = beta_ref[...] - mean * scale
    h = jnp.maximum(h_ref[...] * scale + shift, 0.0)
    h2 = jnp.maximum(
        jnp.dot(h, w2_ref[...], preferred_element_type=jnp.float32)
        + b2_ref[...], 0.0)
    z = jnp.dot(h2, w3_ref[...], preferred_element_type=jnp.float32) + b3_ref[...]
    out_ref[...] = jax.nn.sigmoid(z)


def _tc_mlp(pre, x, w1x, gamma, beta, w2, b2, w3, b3):
    row_blk = lambda i: (i, 0)
    fixed = lambda i: (0, 0)
    h, stats = pl.pallas_call(
        _pass1_body,
        grid=(N_BLK,),
        in_specs=[
            pl.BlockSpec((BLK, HID), row_blk),
            pl.BlockSpec((BLK, NUM_FEATURES), row_blk),
            pl.BlockSpec((NUM_FEATURES, HID), fixed),
        ],
        out_specs=[
            pl.BlockSpec((BLK, HID), row_blk),
            pl.BlockSpec((2, HID), fixed),
        ],
        out_shape=[
            jax.ShapeDtypeStruct((BATCH, HID), jnp.float32),
            jax.ShapeDtypeStruct((2, HID), jnp.float32),
        ],
    )(pre, x, w1x)
    return pl.pallas_call(
        _pass2_body,
        grid=(N_BLK,),
        in_specs=[
            pl.BlockSpec((BLK, HID), row_blk),
            pl.BlockSpec((2, HID), fixed),
            pl.BlockSpec((1, HID), fixed),
            pl.BlockSpec((1, HID), fixed),
            pl.BlockSpec((HID, 64), fixed),
            pl.BlockSpec((1, 64), fixed),
            pl.BlockSpec((64, 1), fixed),
            pl.BlockSpec((1, 1), fixed),
        ],
        out_specs=pl.BlockSpec((BLK, 1), row_blk),
        out_shape=jax.ShapeDtypeStruct((BATCH, 1), jnp.float32),
    )(h, stats, gamma, beta, w2, b2, w3, b3)


def kernel(gene_id, disease_id, explicit_features, gene_table, disease_table,
           W1, b1, gamma, beta, W2, b2, W3, b3):
    gid = gene_id.astype(jnp.int32).reshape(NUM_WORKERS, N_CHUNKS, IDX_CHUNK)
    did = disease_id.astype(jnp.int32).reshape(NUM_WORKERS, N_CHUNKS, IDX_CHUNK)
    pg, pd = _build_products(gene_table, disease_table, W1,
                             b1.reshape(1, -1))
    pre = _sc_gather(pg, pd, gid, did)
    w1x = W1[2 * EMBED_DIM:]
    return _tc_mlp(pre, explicit_features, w1x,
                   gamma.reshape(1, -1), beta.reshape(1, -1),
                   W2, b2.reshape(1, -1), W3, b3.reshape(1, -1))


# speedup vs baseline: 1.9405x; 1.2849x over previous
import functools

import jax
import jax.numpy as jnp
from jax import lax
from jax.experimental import pallas as pl
from jax.experimental.pallas import tpu as pltpu
from jax.experimental.pallas import tpu_sc as plsc

BATCH = 16384
EMBED_DIM = 64
NUM_FEATURES = 128
NUM_GENES = 100000
NUM_DISEASES = 1000
HID = 128

NUM_CORES = 2
NUM_SUBCORES = 16
NUM_WORKERS = NUM_CORES * NUM_SUBCORES
B_PER_W = BATCH // NUM_WORKERS
IDX_CHUNK = 128
N_CHUNKS = B_PER_W // IDX_CHUNK

GBLK = 8192
N_GBLK = -(-NUM_GENES // GBLK)


def _pg_body(tT_ref, w_ref, out_ref):
    out_ref[...] = lax.dot_general(
        tT_ref[...], w_ref[...],
        dimension_numbers=(((0,), (0,)), ((), ())),
        preferred_element_type=jnp.float32)


def _pd_body(tT_ref, w_ref, b_ref, out_ref):
    out_ref[...] = lax.dot_general(
        tT_ref[...], w_ref[...],
        dimension_numbers=(((0,), (0,)), ((), ())),
        preferred_element_type=jnp.float32) + b_ref[...]


def _build_products(gene_table, disease_table, W1, b1):
    gT = jnp.transpose(gene_table)
    dT = jnp.transpose(disease_table)
    w1g = W1[:EMBED_DIM]
    w1d = W1[EMBED_DIM:2 * EMBED_DIM]
    pg = pl.pallas_call(
        _pg_body,
        grid=(N_GBLK,),
        in_specs=[
            pl.BlockSpec((EMBED_DIM, GBLK), lambda i: (0, i)),
            pl.BlockSpec((EMBED_DIM, HID), lambda i: (0, 0)),
        ],
        out_specs=pl.BlockSpec((GBLK, HID), lambda i: (i, 0)),
        out_shape=jax.ShapeDtypeStruct((NUM_GENES, HID), jnp.float32),
    )(gT, w1g)
    pd = pl.pallas_call(
        _pd_body,
        out_shape=jax.ShapeDtypeStruct((NUM_DISEASES, HID), jnp.float32),
    )(dT, w1d, b1)
    return pg, pd


def _gather_body(pg_tab, pd_tab, gid_hbm, did_hbm, pre_out, idx_g, idx_d,
                 rows, sem):
    wid = lax.axis_index("s") * NUM_CORES + lax.axis_index("c")
    base = wid * B_PER_W
    pltpu.sync_copy(gid_hbm.at[wid], idx_g)
    pltpu.sync_copy(did_hbm.at[wid], idx_d)
    copies = []
    for j in range(N_CHUNKS):
        copies.append(pltpu.async_copy(
            pg_tab.at[idx_g.at[j]],
            rows.at[pl.ds(j * IDX_CHUNK, IDX_CHUNK)], sem))
    for c in copies:
        c.wait()
    copies = []
    for j in range(N_CHUNKS):
        copies.append(pltpu.async_copy(
            pd_tab.at[idx_d.at[j]],
            rows.at[pl.ds(j * IDX_CHUNK, IDX_CHUNK)], sem, add=True))
    for c in copies:
        c.wait()
    pltpu.sync_copy(rows, pre_out.at[pl.ds(base, B_PER_W)])


def _sc_gather(pg, pd, gid, did):
    mesh = plsc.VectorSubcoreMesh(core_axis_name="c", subcore_axis_name="s")
    out_type = jax.ShapeDtypeStruct((BATCH, HID), jnp.float32)
    scratch = [
        pltpu.VMEM((N_CHUNKS, IDX_CHUNK), jnp.int32),
        pltpu.VMEM((N_CHUNKS, IDX_CHUNK), jnp.int32),
        pltpu.VMEM((B_PER_W, HID), jnp.float32),
        pltpu.SemaphoreType.DMA,
    ]
    run = pl.kernel(_gather_body, out_type=out_type, mesh=mesh,
                    scratch_types=scratch)
    return run(pg, pd, gid, did)


BLK = 2048
N_BLK = BATCH // BLK


def _pass1_body(pre_ref, x_ref, w1x_ref, h_ref, stats_ref):
    i = pl.program_id(0)
    h = pre_ref[...] + jnp.dot(x_ref[...], w1x_ref[...],
                               preferred_element_type=jnp.float32)
    h_ref[...] = h
    part = jnp.concatenate(
        [jnp.sum(h, axis=0, keepdims=True),
         jnp.sum(h * h, axis=0, keepdims=True)], axis=0)

    @pl.when(i == 0)
    def _():
        stats_ref[...] = part

    @pl.when(i != 0)
    def _():
        stats_ref[...] += part


def _pass2_body(h_ref, stats_ref, gamma_ref, beta_ref, w2_ref, b2_ref,
                w3_ref, b3_ref, out_ref):
    inv_n = 1.0 / BATCH
    mean = stats_ref[0:1, :] * inv_n
    var = stats_ref[1:2, :] * inv_n - mean * mean
    scale = lax.rsqrt(var + 1e-5) * gamma_ref[...]
    shift = beta_ref[...] - mean * scale
    h = jnp.maximum(h_ref[...] * scale + shift, 0.0)
    h2 = jnp.maximum(
        jnp.dot(h, w2_ref[...], preferred_element_type=jnp.float32)
        + b2_ref[...], 0.0)
    z = jnp.dot(h2, w3_ref[...], preferred_element_type=jnp.float32) + b3_ref[...]
    out_ref[...] = jax.nn.sigmoid(z).reshape((BLK,))


def _tc_mlp(pre, x, w1x, gamma, beta, w2, b2, w3, b3):
    row_blk = lambda i: (i, 0)
    fixed = lambda i: (0, 0)
    h, stats = pl.pallas_call(
        _pass1_body,
        grid=(N_BLK,),
        in_specs=[
            pl.BlockSpec((BLK, HID), row_blk),
            pl.BlockSpec((BLK, NUM_FEATURES), row_blk),
            pl.BlockSpec((NUM_FEATURES, HID), fixed),
        ],
        out_specs=[
            pl.BlockSpec((BLK, HID), row_blk),
            pl.BlockSpec((2, HID), fixed),
        ],
        out_shape=[
            jax.ShapeDtypeStruct((BATCH, HID), jnp.float32),
            jax.ShapeDtypeStruct((2, HID), jnp.float32),
        ],
    )(pre, x, w1x)
    return pl.pallas_call(
        _pass2_body,
        grid=(N_BLK,),
        in_specs=[
            pl.BlockSpec((BLK, HID), row_blk),
            pl.BlockSpec((2, HID), fixed),
            pl.BlockSpec((1, HID), fixed),
            pl.BlockSpec((1, HID), fixed),
            pl.BlockSpec((HID, 64), fixed),
            pl.BlockSpec((1, 64), fixed),
            pl.BlockSpec((64, 1), fixed),
            pl.BlockSpec((1, 1), fixed),
        ],
        out_specs=pl.BlockSpec((BLK,), lambda i: (i,)),
        out_shape=jax.ShapeDtypeStruct((BATCH,), jnp.float32),
    )(h, stats, gamma, beta, w2, b2, w3, b3).reshape(BATCH, 1)


def kernel(gene_id, disease_id, explicit_features, gene_table, disease_table,
           W1, b1, gamma, beta, W2, b2, W3, b3):
    gid = gene_id.astype(jnp.int32).reshape(NUM_WORKERS, N_CHUNKS, IDX_CHUNK)
    did = disease_id.astype(jnp.int32).reshape(NUM_WORKERS, N_CHUNKS, IDX_CHUNK)
    pg, pd = _build_products(gene_table, disease_table, W1,
                             b1.reshape(1, -1))
    pre = _sc_gather(pg, pd, gid, did)
    w1x = W1[2 * EMBED_DIM:]
    return _tc_mlp(pre, explicit_features, w1x,
                   gamma.reshape(1, -1), beta.reshape(1, -1),
                   W2, b2.reshape(1, -1), W3, b3.reshape(1, -1))


# speedup vs baseline: 1.9985x; 1.0299x over previous
import functools

import jax
import jax.numpy as jnp
from jax import lax
from jax.experimental import pallas as pl
from jax.experimental.pallas import tpu as pltpu
from jax.experimental.pallas import tpu_sc as plsc

BATCH = 16384
EMBED_DIM = 64
NUM_FEATURES = 128
NUM_GENES = 100000
NUM_DISEASES = 1000
HID = 128

NUM_CORES = 2
NUM_SUBCORES = 16
NUM_WORKERS = NUM_CORES * NUM_SUBCORES
B_PER_W = BATCH // NUM_WORKERS
IDX_CHUNK = 128
N_CHUNKS = B_PER_W // IDX_CHUNK

GBLK = 8192
N_GBLK = -(-NUM_GENES // GBLK)


def _pg_body(tT_ref, w_ref, out_ref):
    out_ref[...] = lax.dot_general(
        tT_ref[...], w_ref[...],
        dimension_numbers=(((0,), (0,)), ((), ())),
        preferred_element_type=jnp.float32)


def _pd_body(tT_ref, w_ref, b_ref, out_ref):
    out_ref[...] = lax.dot_general(
        tT_ref[...], w_ref[...],
        dimension_numbers=(((0,), (0,)), ((), ())),
        preferred_element_type=jnp.float32) + b_ref[...]


def _build_products(gene_table, disease_table, W1, b1):
    gT = jnp.transpose(gene_table)
    dT = jnp.transpose(disease_table)
    w1g = W1[:EMBED_DIM]
    w1d = W1[EMBED_DIM:2 * EMBED_DIM]
    pg = pl.pallas_call(
        _pg_body,
        grid=(N_GBLK,),
        in_specs=[
            pl.BlockSpec((EMBED_DIM, GBLK), lambda i: (0, i)),
            pl.BlockSpec((EMBED_DIM, HID), lambda i: (0, 0)),
        ],
        out_specs=pl.BlockSpec((GBLK, HID), lambda i: (i, 0)),
        out_shape=jax.ShapeDtypeStruct((NUM_GENES, HID), jnp.float32),
    )(gT, w1g)
    pd = pl.pallas_call(
        _pd_body,
        out_shape=jax.ShapeDtypeStruct((NUM_DISEASES, HID), jnp.float32),
    )(dT, w1d, b1)
    return pg, pd


def _gather_body(pg_tab, pd_tab, gid_hbm, did_hbm, pre_out, idx_g, idx_d,
                 rows, sem):
    wid = lax.axis_index("s") * NUM_CORES + lax.axis_index("c")
    base = wid * B_PER_W
    pltpu.sync_copy(gid_hbm.at[wid], idx_g)
    pltpu.sync_copy(did_hbm.at[wid], idx_d)
    copies = []
    for j in range(N_CHUNKS):
        copies.append(pltpu.async_copy(
            pg_tab.at[idx_g.at[j]],
            rows.at[pl.ds(j * IDX_CHUNK, IDX_CHUNK)], sem))
    for c in copies:
        c.wait()
    copies = []
    for j in range(N_CHUNKS):
        copies.append(pltpu.async_copy(
            pd_tab.at[idx_d.at[j]],
            rows.at[pl.ds(j * IDX_CHUNK, IDX_CHUNK)], sem, add=True))
    for c in copies:
        c.wait()
    pltpu.sync_copy(rows, pre_out.at[pl.ds(base, B_PER_W)])


def _sc_gather(pg, pd, gid, did):
    mesh = plsc.VectorSubcoreMesh(core_axis_name="c", subcore_axis_name="s")
    out_type = jax.ShapeDtypeStruct((BATCH, HID), jnp.float32)
    scratch = [
        pltpu.VMEM((N_CHUNKS, IDX_CHUNK), jnp.int32),
        pltpu.VMEM((N_CHUNKS, IDX_CHUNK), jnp.int32),
        pltpu.VMEM((B_PER_W, HID), jnp.float32),
        pltpu.SemaphoreType.DMA,
    ]
    run = pl.kernel(_gather_body, out_type=out_type, mesh=mesh,
                    scratch_types=scratch)
    return run(pg, pd, gid, did)


BLK = 2048
N_BLK = BATCH // BLK


def _mlp_body(pre_ref, x_ref, w1x_ref, gamma_ref, beta_ref, w2_ref, b2_ref,
              w3_ref, b3_ref, out_ref, h_ref, stats_ref):
    s = pl.program_id(0)
    i = s % N_BLK

    @pl.when(s < N_BLK)
    def _phase1():
        h = pre_ref[...] + jnp.dot(x_ref[...], w1x_ref[...],
                                   preferred_element_type=jnp.float32)
        h_ref[pl.ds(i * BLK, BLK), :] = h
        part = jnp.concatenate(
            [jnp.sum(h, axis=0, keepdims=True),
             jnp.sum(h * h, axis=0, keepdims=True)], axis=0)

        @pl.when(s == 0)
        def _():
            stats_ref[...] = part

        @pl.when(s != 0)
        def _():
            stats_ref[...] += part

    @pl.when(s >= N_BLK)
    def _phase2():
        inv_n = 1.0 / BATCH
        mean = stats_ref[0:1, :] * inv_n
        var = stats_ref[1:2, :] * inv_n - mean * mean
        scale = lax.rsqrt(var + 1e-5) * gamma_ref[...]
        shift = beta_ref[...] - mean * scale
        h = jnp.maximum(h_ref[pl.ds(i * BLK, BLK), :] * scale + shift, 0.0)
        h2 = jnp.maximum(
            jnp.dot(h, w2_ref[...], preferred_element_type=jnp.float32)
            + b2_ref[...], 0.0)
        z = (jnp.dot(h2, w3_ref[...], preferred_element_type=jnp.float32)
             + b3_ref[...])
        out_ref[...] = jax.nn.sigmoid(z).reshape((BLK,))


def _tc_mlp(pre, x, w1x, gamma, beta, w2, b2, w3, b3):
    row_blk = lambda s: (s % N_BLK, 0)
    fixed = lambda s: (0, 0)
    return pl.pallas_call(
        _mlp_body,
        grid=(2 * N_BLK,),
        in_specs=[
            pl.BlockSpec((BLK, HID), row_blk),
            pl.BlockSpec((BLK, NUM_FEATURES), row_blk),
            pl.BlockSpec((NUM_FEATURES, HID), fixed),
            pl.BlockSpec((1, HID), fixed),
            pl.BlockSpec((1, HID), fixed),
            pl.BlockSpec((HID, 64), fixed),
            pl.BlockSpec((1, 64), fixed),
            pl.BlockSpec((64, 1), fixed),
            pl.BlockSpec((1, 1), fixed),
        ],
        out_specs=pl.BlockSpec((BLK,), lambda s: (s % N_BLK,)),
        out_shape=jax.ShapeDtypeStruct((BATCH,), jnp.float32),
        scratch_shapes=[
            pltpu.VMEM((BATCH, HID), jnp.float32),
            pltpu.VMEM((2, HID), jnp.float32),
        ],
    )(pre, x, w1x, gamma, beta, w2, b2, w3, b3).reshape(BATCH, 1)


def kernel(gene_id, disease_id, explicit_features, gene_table, disease_table,
           W1, b1, gamma, beta, W2, b2, W3, b3):
    gid = gene_id.astype(jnp.int32).reshape(NUM_WORKERS, N_CHUNKS, IDX_CHUNK)
    did = disease_id.astype(jnp.int32).reshape(NUM_WORKERS, N_CHUNKS, IDX_CHUNK)
    pg, pd = _build_products(gene_table, disease_table, W1,
                             b1.reshape(1, -1))
    pre = _sc_gather(pg, pd, gid, did)
    w1x = W1[2 * EMBED_DIM:]
    return _tc_mlp(pre, explicit_features, w1x,
                   gamma.reshape(1, -1), beta.reshape(1, -1),
                   W2, b2.reshape(1, -1), W3, b3.reshape(1, -1))


# speedup vs baseline: 2.0247x; 1.0131x over previous
import functools

import jax
import jax.numpy as jnp
from jax import lax
from jax.experimental import pallas as pl
from jax.experimental.pallas import tpu as pltpu
from jax.experimental.pallas import tpu_sc as plsc

BATCH = 16384
EMBED_DIM = 64
NUM_FEATURES = 128
NUM_GENES = 100000
NUM_DISEASES = 1000
HID = 128

NUM_CORES = 2
NUM_SUBCORES = 16
NUM_WORKERS = NUM_CORES * NUM_SUBCORES
B_PER_W = BATCH // NUM_WORKERS
IDX_CHUNK = 128
N_CHUNKS = B_PER_W // IDX_CHUNK

GBLK = 8192
N_GBLK = -(-NUM_GENES // GBLK)


def _pg_body(tT_ref, w_ref, out_ref):
    out_ref[...] = lax.dot_general(
        tT_ref[...], w_ref[...],
        dimension_numbers=(((0,), (0,)), ((), ())),
        preferred_element_type=jnp.float32)


def _pd_body(tT_ref, w_ref, b_ref, out_ref):
    out_ref[...] = lax.dot_general(
        tT_ref[...], w_ref[...],
        dimension_numbers=(((0,), (0,)), ((), ())),
        preferred_element_type=jnp.float32) + b_ref[...]


def _build_products(gene_table, disease_table, W1, b1):
    gT = jnp.transpose(gene_table)
    dT = jnp.transpose(disease_table)
    w1g = W1[:EMBED_DIM]
    w1d = W1[EMBED_DIM:2 * EMBED_DIM]
    pg = pl.pallas_call(
        _pg_body,
        grid=(N_GBLK,),
        in_specs=[
            pl.BlockSpec((EMBED_DIM, GBLK), lambda i: (0, i)),
            pl.BlockSpec((EMBED_DIM, HID), lambda i: (0, 0)),
        ],
        out_specs=pl.BlockSpec((GBLK, HID), lambda i: (i, 0)),
        out_shape=jax.ShapeDtypeStruct((NUM_GENES, HID), jnp.float32),
    )(gT, w1g)
    pd = pl.pallas_call(
        _pd_body,
        out_shape=jax.ShapeDtypeStruct((NUM_DISEASES, HID), jnp.float32),
    )(dT, w1d, b1)
    return pg, pd


def _gather_body(pg_tab, pd_tab, gid_hbm, did_hbm, pre_out, idx_g, idx_d,
                 rows, sem):
    wid = lax.axis_index("s") * NUM_CORES + lax.axis_index("c")
    base = wid * B_PER_W
    pltpu.sync_copy(gid_hbm.at[wid], idx_g)
    pltpu.sync_copy(did_hbm.at[wid], idx_d)
    copies = []
    for j in range(N_CHUNKS):
        copies.append(pltpu.async_copy(
            pg_tab.at[idx_g.at[j]],
            rows.at[pl.ds(j * IDX_CHUNK, IDX_CHUNK)], sem))
    for c in copies:
        c.wait()
    copies = []
    for j in range(N_CHUNKS):
        copies.append(pltpu.async_copy(
            pd_tab.at[idx_d.at[j]],
            rows.at[pl.ds(j * IDX_CHUNK, IDX_CHUNK)], sem, add=True))
    for c in copies:
        c.wait()
    pltpu.sync_copy(rows, pre_out.at[pl.ds(base, B_PER_W)])


def _sc_gather(pg, pd, gid, did):
    mesh = plsc.VectorSubcoreMesh(core_axis_name="c", subcore_axis_name="s")
    out_type = jax.ShapeDtypeStruct((BATCH, HID), jnp.float32)
    scratch = [
        pltpu.VMEM((N_CHUNKS, IDX_CHUNK), jnp.int32),
        pltpu.VMEM((N_CHUNKS, IDX_CHUNK), jnp.int32),
        pltpu.VMEM((B_PER_W, HID), jnp.float32),
        pltpu.SemaphoreType.DMA,
    ]
    run = pl.kernel(_gather_body, out_type=out_type, mesh=mesh,
                    scratch_types=scratch)
    return run(pg, pd, gid, did)


BLK = 2048
N_BLK = BATCH // BLK


def _mlp_body(pre_ref, x_ref, w1x_ref, gamma_ref, beta_ref, w2_ref, b2_ref,
              w3_ref, b3_ref, out_ref, h_ref, stats_ref):
    s = pl.program_id(0)
    i = s % N_BLK

    @pl.when(s < N_BLK)
    def _phase1():
        h = pre_ref[...] + jnp.dot(x_ref[...], w1x_ref[...],
                                   preferred_element_type=jnp.float32)
        h_ref[pl.ds(i * BLK, BLK), :] = h
        part = jnp.concatenate(
            [jnp.sum(h, axis=0, keepdims=True),
             jnp.sum(h * h, axis=0, keepdims=True)], axis=0)

        @pl.when(s == 0)
        def _():
            stats_ref[...] = part

        @pl.when(s != 0)
        def _():
            stats_ref[...] += part

    @pl.when(s >= N_BLK)
    def _phase2():
        inv_n = 1.0 / BATCH
        mean = stats_ref[0:1, :] * inv_n
        var = stats_ref[1:2, :] * inv_n - mean * mean
        scale = lax.rsqrt(var + 1e-5) * gamma_ref[...]
        shift = beta_ref[...] - mean * scale
        h = jnp.maximum(h_ref[pl.ds(i * BLK, BLK), :] * scale + shift, 0.0)
        h2 = jnp.maximum(
            jnp.dot(h, w2_ref[...], preferred_element_type=jnp.float32)
            + b2_ref[...], 0.0)
        z = (jnp.dot(h2, w3_ref[...], preferred_element_type=jnp.float32)
             + b3_ref[...])
        out_ref[...] = jax.nn.sigmoid(z).reshape((BLK,))


def _tc_mlp(pre, x, w1x, gamma, beta, w2, b2, w3, b3):
    row_blk = lambda s: (jnp.where(s < N_BLK, s, 0), 0)
    fixed = lambda s: (0, 0)
    return pl.pallas_call(
        _mlp_body,
        grid=(2 * N_BLK,),
        in_specs=[
            pl.BlockSpec((BLK, HID), row_blk),
            pl.BlockSpec((BLK, NUM_FEATURES), row_blk),
            pl.BlockSpec((NUM_FEATURES, HID), fixed),
            pl.BlockSpec((1, HID), fixed),
            pl.BlockSpec((1, HID), fixed),
            pl.BlockSpec((HID, 64), fixed),
            pl.BlockSpec((1, 64), fixed),
            pl.BlockSpec((64, 1), fixed),
            pl.BlockSpec((1, 1), fixed),
        ],
        out_specs=pl.BlockSpec((BLK,), lambda s: (s % N_BLK,)),
        out_shape=jax.ShapeDtypeStruct((BATCH,), jnp.float32),
        scratch_shapes=[
            pltpu.VMEM((BATCH, HID), jnp.float32),
            pltpu.VMEM((2, HID), jnp.float32),
        ],
    )(pre, x, w1x, gamma, beta, w2, b2, w3, b3).reshape(BATCH, 1)


def kernel(gene_id, disease_id, explicit_features, gene_table, disease_table,
           W1, b1, gamma, beta, W2, b2, W3, b3):
    gid = gene_id.astype(jnp.int32).reshape(NUM_WORKERS, N_CHUNKS, IDX_CHUNK)
    did = disease_id.astype(jnp.int32).reshape(NUM_WORKERS, N_CHUNKS, IDX_CHUNK)
    pg, pd = _build_products(gene_table, disease_table, W1,
                             b1.reshape(1, -1))
    pre = _sc_gather(pg, pd, gid, did)
    w1x = W1[2 * EMBED_DIM:]
    return _tc_mlp(pre, explicit_features, w1x,
                   gamma.reshape(1, -1), beta.reshape(1, -1),
                   W2, b2.reshape(1, -1), W3, b3.reshape(1, -1))
